# Initial kernel scaffold; baseline (speedup 1.0000x reference)
import jax
import jax.numpy as jnp
from jax.experimental import pallas as pl


def kernel(x, W):
    raise NotImplementedError("write your pallas kernel here")


# speedup vs baseline: 2.4907x; 2.4907x over previous
import functools

import jax
import jax.numpy as jnp
from jax import lax
from jax.experimental import pallas as pl
from jax.experimental.pallas import tpu as pltpu
from jax.experimental.pallas import tpu_sc as plsc

_D = 16
_NC = 2
_NS = 16
_NW = _NC * _NS
_CH = 2048


def _make_gather(n_total: int):
    per_w = n_total // _NW
    n_ch = per_w // _CH
    mesh = plsc.VectorSubcoreMesh(core_axis_name="c", subcore_axis_name="s")

    @functools.partial(
        pl.kernel,
        mesh=mesh,
        out_type=jax.ShapeDtypeStruct((n_total, _D), jnp.float32),
        scratch_types=[
            pltpu.VMEM((_CH,), jnp.int32),
            pltpu.VMEM((_CH, _D), jnp.float32),
            pltpu.SemaphoreType.DMA,
        ],
        compiler_params=pltpu.CompilerParams(use_tc_tiling_on_sc=False),
    )
    def k(idx_hbm, table_hbm, out_hbm, idx_v, rows_v, sem):
        wid = lax.axis_index("s") * _NC + lax.axis_index("c")
        base = wid * per_w

        def body(c, _):
            off = base + c * _CH
            pltpu.sync_copy(idx_hbm.at[pl.ds(off, _CH)], idx_v)
            pltpu.async_copy(table_hbm.at[idx_v], rows_v, sem).wait()
            pltpu.sync_copy(rows_v, out_hbm.at[pl.ds(off, _CH)])
            return 0

        lax.fori_loop(0, n_ch, body, 0)

    return k


def kernel(x, W):
    b, h = x.shape
    n = b * h
    idx = x.reshape(n).astype(jnp.int32)
    out = _make_gather(n)(idx, W)
    return out.reshape(b, h, _D)


# speedup vs baseline: 2.5340x; 1.0174x over previous
import functools

import jax
import jax.numpy as jnp
from jax import lax
from jax.experimental import pallas as pl
from jax.experimental.pallas import tpu as pltpu
from jax.experimental.pallas import tpu_sc as plsc

_D = 16
_NC = 2
_NS = 16
_NW = _NC * _NS
_CH = 1024
_K = 5
_G = _CH * _K


def _make_gather(n_total: int):
    per_w = n_total // _NW
    n_grp = per_w // _G
    mesh = plsc.VectorSubcoreMesh(core_axis_name="c", subcore_axis_name="s")

    @functools.partial(
        pl.kernel,
        mesh=mesh,
        out_type=jax.ShapeDtypeStruct((n_total, _D), jnp.float32),
        scratch_types=[
            pltpu.VMEM((_G,), jnp.int32),
            pltpu.VMEM((_G, _D), jnp.float32),
            pltpu.SemaphoreType.DMA,
        ],
        compiler_params=pltpu.CompilerParams(use_tc_tiling_on_sc=False),
    )
    def k(idx_hbm, table_hbm, out_hbm, idx_v, rows_v, gsem):
        wid = lax.axis_index("s") * _NC + lax.axis_index("c")
        base = wid * per_w

        def body(g, _):
            off = base + g * _G
            pltpu.sync_copy(idx_hbm.at[pl.ds(off, _G)], idx_v)
            copies = [
                pltpu.async_copy(
                    table_hbm.at[idx_v.at[pl.ds(b * _CH, _CH)]],
                    rows_v.at[pl.ds(b * _CH, _CH)], gsem)
                for b in range(_K)
            ]
            for c in copies:
                c.wait()
            pltpu.sync_copy(rows_v, out_hbm.at[pl.ds(off, _G)])
            return 0

        lax.fori_loop(0, n_grp, body, 0)

    return k


def kernel(x, W):
    b, h = x.shape
    n = b * h
    idx = x.reshape(n).astype(jnp.int32)
    out = _make_gather(n)(idx, W)
    return out.reshape(b, h, _D)


# speedup vs baseline: 3.6217x; 1.4293x over previous
import functools

import jax
import jax.numpy as jnp
from jax import lax
from jax.experimental import pallas as pl
from jax.experimental.pallas import tpu as pltpu
from jax.experimental.pallas import tpu_sc as plsc

_B = 16384
_H = 200
_D = 16
_NC = 2
_NS = 16
_NW = _NC * _NS
_CH = 2048
_NBC = _B // _CH
_UNITS = _H * _NBC
_PER_W = _UNITS // _NW
_TSZ = _CH * _D


def _make_gather():
    mesh = plsc.VectorSubcoreMesh(core_axis_name="c", subcore_axis_name="s")

    @functools.partial(
        pl.kernel,
        mesh=mesh,
        out_type=jax.ShapeDtypeStruct((_H, 2, _B * 8), jnp.float32),
        scratch_types=[
            pltpu.VMEM((_CH,), jnp.int32),
            pltpu.VMEM((_CH, _D), jnp.float32),
            pltpu.VMEM((_TSZ,), jnp.float32),
            pltpu.SemaphoreType.DMA,
        ],
        compiler_params=pltpu.CompilerParams(use_tc_tiling_on_sc=False, needs_layout_passes=False),
    )
    def k(xt_hbm, w_hbm, out_hbm, idx_v, rows_v, t_v, sem):
        wid = lax.axis_index("s") * _NC + lax.axis_index("c")
        lanes = lax.iota(jnp.int32, 16)
        pos0 = (lanes // 8) * (_CH * 8) + (lanes % 8) * 128

        def unit(j, _):
            u = wid * _PER_W + j
            h = u // _NBC
            bcb = u % _NBC
            pltpu.sync_copy(xt_hbm.at[h, pl.ds(bcb * _CH, _CH)], idx_v)
            pltpu.async_copy(w_hbm.at[idx_v], rows_v, sem).wait()

            def block(r0, _):
                base = (r0 // 8) * 1024 + (r0 % 8) * 16
                for i in range(16):
                    row = rows_v[r0 * 16 + i, :]
                    plsc.store_scatter(t_v, [pos0 + (base + i)], row)
                return 0

            lax.fori_loop(0, _CH // 16, block, 0)
            for dh in range(2):
                pltpu.sync_copy(
                    t_v.at[pl.ds(dh * (_CH * 8), _CH * 8)],
                    out_hbm.at[h, dh, pl.ds(bcb * _CH * 8, _CH * 8)])
            return 0

        lax.fori_loop(0, _PER_W, unit, 0)

    return k


def kernel(x, W):
    x_t = jnp.swapaxes(x, 0, 1).astype(jnp.int32)
    out5 = _make_gather()(x_t, W)
    t = out5.reshape(_H, 2, _B // 128, 8, 128)
    t = t.transpose(0, 1, 3, 2, 4)
    t = t.reshape(_H, _D, _B)
    return t.transpose(2, 0, 1)
